# baseline (speedup 1.0000x reference)
import functools

import jax
import jax.numpy as jnp
from jax import lax
from jax.experimental import pallas as pl
from jax.experimental.pallas import tpu as pltpu
from jax.experimental.pallas import tpu_sc as plsc

N = 10000
D_IN = 128
D_OUT = 64
K_HOPS = 3
E = 320000

NC = 2
NS = 16
CH = D_OUT // NC
RT = 632
CHUNK = 128
NPASS = 4
PC = 40
NCHUNK = NPASS * PC
EP = NS * NCHUNK * CHUNK
NPAD = 10016
DUMMY = N


def _tc_matmul_body(x_ref, w1_ref, w2_ref, b1_ref, b2_ref, xp_ref, bc_ref):
    w = jax.lax.dot_general(w1_ref[...], w2_ref[...],
                            (((1,), (0,)), ((), ())),
                            preferred_element_type=jnp.float32)
    h = jax.lax.dot_general(x_ref[...], w,
                            (((1,), (0,)), ((), ())),
                            preferred_element_type=jnp.float32)
    xp_ref[0] = h[:, :CH]
    xp_ref[1] = h[:, CH:]
    bc_ref[...] = jax.lax.dot_general(
        b1_ref[...], w2_ref[...], (((1,), (0,)), ((), ())),
        preferred_element_type=jnp.float32) + b2_ref[...]


def _tc_matmul(x, W1, b1, W2, b2):
    blk = 1000
    grid = (N // blk,)
    return pl.pallas_call(
        _tc_matmul_body,
        grid=grid,
        in_specs=[
            pl.BlockSpec((blk, D_IN), lambda i: (i, 0)),
            pl.BlockSpec((D_IN, D_IN), lambda i: (0, 0)),
            pl.BlockSpec((D_IN, D_OUT), lambda i: (0, 0)),
            pl.BlockSpec((1, D_IN), lambda i: (0, 0)),
            pl.BlockSpec((1, D_OUT), lambda i: (0, 0)),
        ],
        out_specs=[
            pl.BlockSpec((NC, blk, CH), lambda i: (0, i, 0)),
            pl.BlockSpec((1, D_OUT), lambda i: (0, 0)),
        ],
        out_shape=[
            jax.ShapeDtypeStruct((NC, N, CH), jnp.float32),
            jax.ShapeDtypeStruct((1, D_OUT), jnp.float32),
        ],
    )(x, W1, W2, b1.reshape(1, D_IN), b2.reshape(1, D_OUT))


def _newton_rsqrt(x):
    i = plsc.bitcast(x, jnp.int32)
    i = jnp.int32(0x5F3759DF) - jnp.right_shift(i, 1)
    y = plsc.bitcast(i, jnp.float32)
    for _ in range(3):
        y = y * (1.5 - 0.5 * x * y * y)
    return y


def _sc_body(xp_hbm, src_hbm, dst_hbm, bc_hbm, out_hbm,
             src_t, dst_t, rowbuf, abuf,
             gbuf0, gbuf1, gbuf2, gbuf3, dinv_b, ones_t, bc_t,
             gsem0, gsem1, gsem2, gsem3, ssem,
             g_sp, acc_sp, deg_sp):
    c = lax.axis_index("c")
    t = lax.axis_index("s")
    rbase = pl.multiple_of(jnp.where(t < NS - 1, t * RT, N - RT), 8)

    zvec = jnp.zeros((16,), jnp.float32)
    ovec = jnp.ones((16,), jnp.float32)
    gbufs = [gbuf0, gbuf1, gbuf2, gbuf3]
    gsems = [gsem0, gsem1, gsem2, gsem3]

    pltpu.sync_copy(bc_hbm.at[c], bc_t)

    def stage_edges(p):
        pltpu.sync_copy(src_hbm.at[t, pl.ds(p * PC, PC)], src_t)
        pltpu.sync_copy(dst_hbm.at[t, pl.ds(p * PC, PC)], dst_t)

    def z16(i, _):
        dinv_b[i, pl.ds(0, 16)] = zvec
        ones_t[i % CHUNK, pl.ds(0, 16)] = ovec
        return 0
    lax.fori_loop(0, 640, z16, 0)
    pltpu.sync_copy(dinv_b.at[pl.ds(0, RT)], deg_sp.at[pl.ds(rbase, RT)])

    @pl.when(t == 0)
    def _():
        pltpu.sync_copy(dinv_b.at[pl.ds(0, 16)], deg_sp.at[pl.ds(N, 16)])

    plsc.subcore_barrier()

    for p in range(NPASS):
        stage_edges(p)

        def degstep(j, _):
            pltpu.sync_copy(ones_t, deg_sp.at[dst_t.at[j]], add=True)
            return 0
        lax.fori_loop(0, PC, degstep, 0)

    plsc.subcore_barrier()

    pltpu.sync_copy(deg_sp.at[pl.ds(rbase, RT)], dinv_b.at[pl.ds(0, RT)])

    def dloop(i, _):
        d = dinv_b[i, pl.ds(0, 16)] + 1.0
        dinv_b[i, pl.ds(0, 16)] = _newton_rsqrt(d)
        return 0
    lax.fori_loop(0, RT, dloop, 0)

    pltpu.sync_copy(xp_hbm.at[c, pl.ds(rbase, RT)], rowbuf)

    def initloop(i, _):
        s = dinv_b[i, pl.ds(0, 16)]
        for h in range(2):
            v = rowbuf[i, pl.ds(h * 16, 16)] * s
            rowbuf[i, pl.ds(h * 16, 16)] = v
            abuf[i, pl.ds(h * 16, 16)] = zvec
        return 0
    lax.fori_loop(0, RT, initloop, 0)
    pltpu.sync_copy(rowbuf, g_sp.at[pl.ds(rbase, RT)])
    pltpu.sync_copy(abuf, acc_sp.at[pl.ds(rbase, RT)])

    @pl.when(t == 0)
    def _():
        pltpu.sync_copy(abuf.at[pl.ds(0, 16)], g_sp.at[pl.ds(N, 16)])
        pltpu.sync_copy(abuf.at[pl.ds(0, 16)], acc_sp.at[pl.ds(N, 16)])

    plsc.subcore_barrier()

    for hop in range(K_HOPS):
        last = hop == K_HOPS - 1

        for p in range(NPASS):
            stage_edges(p)
            for s in range(3):
                pltpu.async_copy(g_sp.at[src_t.at[s]], gbufs[s], gsems[s])

            def quad(jj, _):
                j0 = jj * 4
                for k in range(4):
                    j = j0 + k
                    s = k
                    s3 = (k + 3) % 4
                    pltpu.make_async_copy(
                        g_sp.at[src_t.at[0]], gbufs[s], gsems[s]).wait()
                    @pl.when(j > 0)
                    def _():
                        pltpu.make_async_copy(
                            gbufs[s], acc_sp.at[dst_t.at[0]], ssem).wait()
                    pltpu.async_copy(gbufs[s], acc_sp.at[dst_t.at[j]],
                                    ssem, add=True)

                    @pl.when(j + 3 < PC)
                    def _():
                        pltpu.async_copy(g_sp.at[src_t.at[j + 3]],
                                         gbufs[s3], gsems[s3])
                return 0
            lax.fori_loop(0, PC // 4, quad, 0)
            pltpu.make_async_copy(
                gbufs[0], acc_sp.at[dst_t.at[0]], ssem).wait()

        plsc.subcore_barrier()

        pltpu.sync_copy(acc_sp.at[pl.ds(rbase, RT)], abuf)
        pltpu.sync_copy(g_sp.at[pl.ds(rbase, RT)], rowbuf)

        if not last:
            def sloop(i, _):
                s = dinv_b[i, pl.ds(0, 16)]
                for h in range(2):
                    v = (abuf[i, pl.ds(h * 16, 16)]
                         + rowbuf[i, pl.ds(h * 16, 16)]) * s * s
                    rowbuf[i, pl.ds(h * 16, 16)] = v
                    abuf[i, pl.ds(h * 16, 16)] = zvec
                return 0
            lax.fori_loop(0, RT, sloop, 0)
            pltpu.sync_copy(rowbuf, g_sp.at[pl.ds(rbase, RT)])
            pltpu.sync_copy(abuf, acc_sp.at[pl.ds(rbase, RT)])
            plsc.subcore_barrier()
        else:
            bvecs = [bc_t[pl.ds(0, 16)], bc_t[pl.ds(16, 16)]]

            def floop(i, _):
                s = dinv_b[i, pl.ds(0, 16)]
                for h in range(2):
                    v = (abuf[i, pl.ds(h * 16, 16)]
                         + rowbuf[i, pl.ds(h * 16, 16)]) * s + bvecs[h]
                    rowbuf[i, pl.ds(h * 16, 16)] = v
                return 0
            lax.fori_loop(0, RT, floop, 0)
            pltpu.sync_copy(rowbuf, out_hbm.at[c, pl.ds(rbase, RT)])


@functools.partial(jax.jit, static_argnames=())
def _sc_propagate(xp, src3, dst3, bc):
    mesh = plsc.VectorSubcoreMesh(core_axis_name="c", subcore_axis_name="s")
    f = pl.kernel(
        _sc_body,
        out_type=jax.ShapeDtypeStruct((NC, N, CH), jnp.float32),
        mesh=mesh,
        compiler_params=pltpu.CompilerParams(
            needs_layout_passes=False, use_tc_tiling_on_sc=False),
        scratch_types=[
            pltpu.VMEM((PC, CHUNK), jnp.int32),
            pltpu.VMEM((PC, CHUNK), jnp.int32),
            pltpu.VMEM((RT, CH), jnp.float32),
            pltpu.VMEM((RT, CH), jnp.float32),
            pltpu.VMEM((CHUNK, CH), jnp.float32),
            pltpu.VMEM((CHUNK, CH), jnp.float32),
            pltpu.VMEM((CHUNK, CH), jnp.float32),
            pltpu.VMEM((CHUNK, CH), jnp.float32),
            pltpu.VMEM((640, 16), jnp.float32),
            pltpu.VMEM((CHUNK, 16), jnp.float32),
            pltpu.VMEM((CH,), jnp.float32),
            pltpu.SemaphoreType.DMA,
            pltpu.SemaphoreType.DMA,
            pltpu.SemaphoreType.DMA,
            pltpu.SemaphoreType.DMA,
            pltpu.SemaphoreType.DMA,
            pltpu.VMEM_SHARED((NPAD, CH), jnp.float32),
            pltpu.VMEM_SHARED((NPAD, CH), jnp.float32),
            pltpu.VMEM_SHARED((NPAD, 16), jnp.float32),
        ],
    )
    return f(xp, src3, dst3, bc)


def kernel(x, edge_index, W1, b1, W2, b2):
    xp, bc = _tc_matmul(x, W1, b1, W2, b2)
    bc = bc.reshape(NC, CH)
    src = edge_index[0].astype(jnp.int32)
    dst = edge_index[1].astype(jnp.int32)
    pad = EP - E
    src3 = jnp.concatenate(
        [src, jnp.zeros((pad,), jnp.int32)]).reshape(NS, NCHUNK, CHUNK)
    dst3 = jnp.concatenate(
        [dst, jnp.full((pad,), DUMMY, jnp.int32)]).reshape(NS, NCHUNK, CHUNK)
    out2 = _sc_propagate(xp, src3, dst3, bc)
    return out2.transpose(1, 0, 2).reshape(N, D_OUT)


# speedup vs baseline: 1.0675x; 1.0675x over previous
import functools

import jax
import jax.numpy as jnp
from jax import lax
from jax.experimental import pallas as pl
from jax.experimental.pallas import tpu as pltpu
from jax.experimental.pallas import tpu_sc as plsc

N = 10000
D_IN = 128
D_OUT = 64
K_HOPS = 3
E = 320000

NC = 2
NS = 16
CH = D_OUT // NC
RT = 640
CHUNK = 128
NPASS = 4
PC = 40
NCHUNK = NPASS * PC
EP = NS * NCHUNK * CHUNK
NPAD = 10016
DUMMY = N


def _tc_matmul_body(x_ref, w1_ref, w2_ref, b1_ref, b2_ref, xp_ref, bc_ref):
    w = jax.lax.dot_general(w1_ref[...], w2_ref[...],
                            (((1,), (0,)), ((), ())),
                            preferred_element_type=jnp.float32)
    h = jax.lax.dot_general(x_ref[...], w,
                            (((1,), (0,)), ((), ())),
                            preferred_element_type=jnp.float32)
    xp_ref[0] = h[:, :CH]
    xp_ref[1] = h[:, CH:]
    bc_ref[...] = jax.lax.dot_general(
        b1_ref[...], w2_ref[...], (((1,), (0,)), ((), ())),
        preferred_element_type=jnp.float32) + b2_ref[...]


def _tc_matmul(x, W1, b1, W2, b2):
    blk = 1000
    grid = (N // blk,)
    return pl.pallas_call(
        _tc_matmul_body,
        grid=grid,
        in_specs=[
            pl.BlockSpec((blk, D_IN), lambda i: (i, 0)),
            pl.BlockSpec((D_IN, D_IN), lambda i: (0, 0)),
            pl.BlockSpec((D_IN, D_OUT), lambda i: (0, 0)),
            pl.BlockSpec((1, D_IN), lambda i: (0, 0)),
            pl.BlockSpec((1, D_OUT), lambda i: (0, 0)),
        ],
        out_specs=[
            pl.BlockSpec((NC, blk, CH), lambda i: (0, i, 0)),
            pl.BlockSpec((1, D_OUT), lambda i: (0, 0)),
        ],
        out_shape=[
            jax.ShapeDtypeStruct((NC, N, CH), jnp.float32),
            jax.ShapeDtypeStruct((1, D_OUT), jnp.float32),
        ],
    )(x, W1, W2, b1.reshape(1, D_IN), b2.reshape(1, D_OUT))


def _newton_rsqrt(x):
    i = plsc.bitcast(x, jnp.int32)
    i = jnp.int32(0x5F3759DF) - jnp.right_shift(i, 1)
    y = plsc.bitcast(i, jnp.float32)
    for _ in range(3):
        y = y * (1.5 - 0.5 * x * y * y)
    return y


def _sc_body(xp_hbm, src_hbm, dst_hbm, bc_hbm, out_hbm,
             src_t, dst_t, rowbuf, abuf,
             gbuf0, gbuf1, gbuf2, gbuf3, hist, idbuf, dinv_t, dinv2_t, bc_t,
             gsem0, gsem1, gsem2, gsem3, ssem,
             g_sp, acc_sp, deg_sp):
    c = lax.axis_index("c")
    t = lax.axis_index("s")
    rbase = pl.multiple_of(jnp.where(t < NS - 1, t * RT, N - RT), 8)
    drow = pl.multiple_of(jnp.where(t < NS - 1, t * (RT // 16), 584), 8)
    doff = jnp.where(t < NS - 1, 0, 16)

    zvec = jnp.zeros((16,), jnp.float32)
    ovec = jnp.ones((16,), jnp.float32)
    zlanes = jnp.zeros((16,), jnp.int32)
    gbufs = [gbuf0, gbuf1, gbuf2, gbuf3]
    gsems = [gsem0, gsem1, gsem2, gsem3]

    def bcast16(ref, i):
        return plsc.load_gather(ref, [zlanes + i])

    pltpu.sync_copy(bc_hbm.at[c], bc_t)

    def stage_edges(p):
        pltpu.sync_copy(src_hbm.at[t, pl.ds(p * PC, PC)], src_t)
        pltpu.sync_copy(dst_hbm.at[t, pl.ds(p * PC, PC)], dst_t)

    iota16 = jnp.arange(16, dtype=jnp.int32)

    def zh(i, _):
        hist[i, pl.ds(0, 16)] = zvec
        return 0
    lax.fori_loop(0, 640, zh, 0)
    zrow = pl.multiple_of(t * 40, 8)
    pltpu.sync_copy(hist.at[pl.ds(zrow, 40)], deg_sp.at[pl.ds(zrow, 40)])
    for q in range(5):
        for l in range(8):
            idbuf[q, pl.ds(l * 16, 16)] = q * 128 + l * 16 + iota16

    plsc.subcore_barrier()

    for p in range(NPASS):
        stage_edges(p)

        def degstep(j, _):
            for l in range(CHUNK // 16):
                dstv = dst_t[j, pl.ds(l * 16, 16)]
                plsc.addupdate_scatter(
                    hist, [jnp.right_shift(dstv, 4),
                           jnp.bitwise_and(dstv, 15)], ovec)
            return 0
        lax.fori_loop(0, PC, degstep, 0)
    for q in range(5):
        pltpu.sync_copy(hist.at[pl.ds(q * 128, 128)],
                        deg_sp.at[idbuf.at[q]], add=True)

    plsc.subcore_barrier()

    pltpu.sync_copy(deg_sp.at[pl.ds(drow, 48)], hist.at[pl.ds(0, 48)])

    def dloop(i, _):
        d = hist[i, pl.ds(0, 16)] + 1.0
        y = _newton_rsqrt(d)
        dinv_t[pl.ds(i * 16, 16)] = y
        dinv2_t[pl.ds(i * 16, 16)] = y * y
        return 0
    lax.fori_loop(0, 48, dloop, 0)

    pltpu.sync_copy(xp_hbm.at[c, pl.ds(rbase, RT)], rowbuf)

    def initloop(i, _):
        s = bcast16(dinv_t, i + doff)
        for h in range(2):
            v = rowbuf[i, pl.ds(h * 16, 16)] * s
            rowbuf[i, pl.ds(h * 16, 16)] = v
            abuf[i, pl.ds(h * 16, 16)] = zvec
        return 0
    lax.fori_loop(0, RT, initloop, 0)
    pltpu.sync_copy(rowbuf, g_sp.at[pl.ds(rbase, RT)])
    pltpu.sync_copy(abuf, acc_sp.at[pl.ds(rbase, RT)])

    @pl.when(t == 0)
    def _():
        pltpu.sync_copy(abuf.at[pl.ds(0, 16)], g_sp.at[pl.ds(N, 16)])
        pltpu.sync_copy(abuf.at[pl.ds(0, 16)], acc_sp.at[pl.ds(N, 16)])

    plsc.subcore_barrier()

    for hop in range(K_HOPS):
        last = hop == K_HOPS - 1

        for p in range(NPASS):
            stage_edges(p)
            for s in range(3):
                pltpu.async_copy(g_sp.at[src_t.at[s]], gbufs[s], gsems[s])

            def quad(jj, _):
                j0 = jj * 4
                for k in range(4):
                    j = j0 + k
                    s = k
                    s3 = (k + 3) % 4
                    pltpu.make_async_copy(
                        g_sp.at[src_t.at[0]], gbufs[s], gsems[s]).wait()
                    @pl.when(j > 0)
                    def _():
                        pltpu.make_async_copy(
                            gbufs[s], acc_sp.at[dst_t.at[0]], ssem).wait()
                    pltpu.async_copy(gbufs[s], acc_sp.at[dst_t.at[j]],
                                    ssem, add=True)

                    @pl.when(j + 3 < PC)
                    def _():
                        pltpu.async_copy(g_sp.at[src_t.at[j + 3]],
                                         gbufs[s3], gsems[s3])
                return 0
            lax.fori_loop(0, PC // 4, quad, 0)
            pltpu.make_async_copy(
                gbufs[0], acc_sp.at[dst_t.at[0]], ssem).wait()

        plsc.subcore_barrier()

        pltpu.sync_copy(acc_sp.at[pl.ds(rbase, RT)], abuf)
        pltpu.sync_copy(g_sp.at[pl.ds(rbase, RT)], rowbuf)

        if not last:
            def sloop(i, _):
                s2 = bcast16(dinv2_t, i + doff)
                for h in range(2):
                    v = (abuf[i, pl.ds(h * 16, 16)]
                         + rowbuf[i, pl.ds(h * 16, 16)]) * s2
                    rowbuf[i, pl.ds(h * 16, 16)] = v
                    abuf[i, pl.ds(h * 16, 16)] = zvec
                return 0
            lax.fori_loop(0, RT, sloop, 0)
            pltpu.sync_copy(rowbuf, g_sp.at[pl.ds(rbase, RT)])
            pltpu.sync_copy(abuf, acc_sp.at[pl.ds(rbase, RT)])
            plsc.subcore_barrier()
        else:
            bvecs = [bc_t[pl.ds(0, 16)], bc_t[pl.ds(16, 16)]]

            def floop(i, _):
                s = bcast16(dinv_t, i + doff)
                for h in range(2):
                    v = (abuf[i, pl.ds(h * 16, 16)]
                         + rowbuf[i, pl.ds(h * 16, 16)]) * s + bvecs[h]
                    rowbuf[i, pl.ds(h * 16, 16)] = v
                return 0
            lax.fori_loop(0, RT, floop, 0)
            pltpu.sync_copy(rowbuf, out_hbm.at[c, pl.ds(rbase, RT)])


@functools.partial(jax.jit, static_argnames=())
def _sc_propagate(xp, src3, dst3, bc):
    mesh = plsc.VectorSubcoreMesh(core_axis_name="c", subcore_axis_name="s")
    f = pl.kernel(
        _sc_body,
        out_type=jax.ShapeDtypeStruct((NC, N, CH), jnp.float32),
        mesh=mesh,
        compiler_params=pltpu.CompilerParams(
            needs_layout_passes=False, use_tc_tiling_on_sc=False),
        scratch_types=[
            pltpu.VMEM((PC, CHUNK), jnp.int32),
            pltpu.VMEM((PC, CHUNK), jnp.int32),
            pltpu.VMEM((RT, CH), jnp.float32),
            pltpu.VMEM((RT, CH), jnp.float32),
            pltpu.VMEM((CHUNK, CH), jnp.float32),
            pltpu.VMEM((CHUNK, CH), jnp.float32),
            pltpu.VMEM((CHUNK, CH), jnp.float32),
            pltpu.VMEM((CHUNK, CH), jnp.float32),
            pltpu.VMEM((640, 16), jnp.float32),
            pltpu.VMEM((5, CHUNK), jnp.int32),
            pltpu.VMEM((768,), jnp.float32),
            pltpu.VMEM((768,), jnp.float32),
            pltpu.VMEM((CH,), jnp.float32),
            pltpu.SemaphoreType.DMA,
            pltpu.SemaphoreType.DMA,
            pltpu.SemaphoreType.DMA,
            pltpu.SemaphoreType.DMA,
            pltpu.SemaphoreType.DMA,
            pltpu.VMEM_SHARED((NPAD, CH), jnp.float32),
            pltpu.VMEM_SHARED((NPAD, CH), jnp.float32),
            pltpu.VMEM_SHARED((640, 16), jnp.float32),
        ],
    )
    return f(xp, src3, dst3, bc)


def kernel(x, edge_index, W1, b1, W2, b2):
    xp, bc = _tc_matmul(x, W1, b1, W2, b2)
    bc = bc.reshape(NC, CH)
    src = edge_index[0].astype(jnp.int32)
    dst = edge_index[1].astype(jnp.int32)
    pad = EP - E
    src3 = jnp.concatenate(
        [src, jnp.zeros((pad,), jnp.int32)]).reshape(NS, NCHUNK, CHUNK)
    dst3 = jnp.concatenate(
        [dst, jnp.full((pad,), DUMMY, jnp.int32)]).reshape(NS, NCHUNK, CHUNK)
    out2 = _sc_propagate(xp, src3, dst3, bc)
    return out2.transpose(1, 0, 2).reshape(N, D_OUT)
